# baseline (speedup 1.0000x reference)
import functools

import jax
import jax.numpy as jnp
from jax import lax
from jax.experimental import pallas as pl
from jax.experimental.pallas import tpu as pltpu
from jax.experimental.pallas import tpu_sc as plsc

_NUM_CORES = 2
_NUM_SUBCORES = 16
_NW = _NUM_CORES * _NUM_SUBCORES


@functools.cache
def _build_gather(batch: int, dim: int):
    n_pairs = batch // 2
    p_per_w = n_pairs // _NW
    mesh = plsc.VectorSubcoreMesh(core_axis_name="c", subcore_axis_name="s")

    @functools.partial(
        pl.kernel,
        mesh=mesh,
        out_type=jax.ShapeDtypeStruct((n_pairs, 2 * dim), jnp.float32),
        scratch_types=[
            pltpu.VMEM((p_per_w,), jnp.int32),
            pltpu.VMEM((p_per_w,), jnp.int32),
            pltpu.VMEM((p_per_w, dim), jnp.float32),
            pltpu.VMEM((p_per_w, dim), jnp.float32),
            pltpu.SemaphoreType.DMA,
            pltpu.SemaphoreType.DMA,
        ],
        compiler_params=pltpu.CompilerParams(use_tc_tiling_on_sc=False),
    )
    def gather_kernel(
        idx_e_hbm, idx_o_hbm, table_hbm, out_hbm,
        idx_e, idx_o, rows_e, rows_o, sem, sem2,
    ):
        wid = lax.axis_index("s") * _NUM_CORES + lax.axis_index("c")
        base = wid * p_per_w
        pltpu.sync_copy(idx_e_hbm.at[pl.ds(base, p_per_w)], idx_e)
        pltpu.sync_copy(idx_o_hbm.at[pl.ds(base, p_per_w)], idx_o)
        c1 = pltpu.async_copy(table_hbm.at[idx_e], rows_e, sem)
        c2 = pltpu.async_copy(table_hbm.at[idx_o], rows_o, sem2)
        c1.wait()
        c2.wait()
        pltpu.sync_copy(
            rows_e, out_hbm.at[pl.ds(base, p_per_w), pl.ds(0, dim)]
        )
        pltpu.sync_copy(
            rows_o, out_hbm.at[pl.ds(base, p_per_w), pl.ds(dim, dim)]
        )

    return gather_kernel


def kernel(labels, table):
    labels = labels.astype(jnp.int32)
    batch = labels.shape[0]
    dim = table.shape[1]
    table = table.astype(jnp.float32)
    packed = _build_gather(batch, dim)(labels[0::2], labels[1::2], table)
    return packed.reshape(batch, dim)


# speedup vs baseline: 1.7303x; 1.7303x over previous
import functools

import jax
import jax.numpy as jnp
from jax import lax
from jax.experimental import pallas as pl
from jax.experimental.pallas import tpu as pltpu
from jax.experimental.pallas import tpu_sc as plsc

_NUM_CORES = 2
_NUM_SUBCORES = 16
_NW = _NUM_CORES * _NUM_SUBCORES
_LANES = 16


@functools.cache
def _build_gather(batch: int, dim: int):
    b_per_w = batch // _NW
    n_groups = b_per_w // _LANES
    mesh = plsc.VectorSubcoreMesh(core_axis_name="c", subcore_axis_name="s")

    @functools.partial(
        pl.kernel,
        mesh=mesh,
        out_type=jax.ShapeDtypeStruct((batch, dim), jnp.float32),
        scratch_types=[
            pltpu.VMEM((b_per_w,), jnp.int32),
            pltpu.VMEM((b_per_w, dim), jnp.float32),
            pltpu.SemaphoreType.DMA,
        ],
    )
    def gather_kernel(idx_hbm, table_hbm, out_hbm, idx_v, rows_v, sem):
        wid = lax.axis_index("s") * _NUM_CORES + lax.axis_index("c")
        base = wid * b_per_w
        pltpu.sync_copy(idx_hbm.at[pl.ds(base, b_per_w)], idx_v)

        @plsc.parallel_loop(0, n_groups, 1, unroll=4)
        def _(g):
            vec = idx_v[pl.ds(g * _LANES, _LANES)]
            for l in range(_LANES):
                r = vec[l]
                pltpu.async_copy(
                    table_hbm.at[r], rows_v.at[g * _LANES + l], sem
                )

        pltpu.make_async_copy(
            table_hbm.at[pl.ds(0, b_per_w)], rows_v, sem
        ).wait()
        pltpu.sync_copy(rows_v, out_hbm.at[pl.ds(base, b_per_w)])

    return gather_kernel


def kernel(labels, table):
    labels = labels.astype(jnp.int32)
    batch = labels.shape[0]
    dim = table.shape[1]
    table = table.astype(jnp.float32)
    return _build_gather(batch, dim)(labels, table)
